# Initial kernel scaffold; baseline (speedup 1.0000x reference)
import jax
import jax.numpy as jnp
from jax.experimental import pallas as pl


def kernel(x, edge_index, W1, b1, W2, b2):
    raise NotImplementedError("write your pallas kernel here")


# speedup vs baseline: 1.0827x; 1.0827x over previous
import jax
import jax.numpy as jnp
from jax.experimental import pallas as pl


def _bias_kernel(a_ref, b_ref, o_ref):
    o_ref[...] = a_ref[...] + b_ref[...]


def kernel(x, edge_index, W1, b1, W2, b2):
    num_nodes = x.shape[0]
    loop = jnp.arange(num_nodes, dtype=edge_index.dtype)
    src = jnp.concatenate([edge_index[0], loop])
    dst = jnp.concatenate([edge_index[1], loop])
    deg = jnp.zeros((num_nodes,), dtype=jnp.float32).at[dst].add(1.0)
    deg_inv_sqrt = jnp.where(deg > 0, jax.lax.rsqrt(deg), 0.0)
    norm = deg_inv_sqrt[src] * deg_inv_sqrt[dst]

    def conv(h, W, b):
        h = h @ W
        msgs = jnp.take(h, src, axis=0) * norm[:, None]
        out = jnp.zeros((num_nodes, h.shape[1]), dtype=h.dtype).at[dst].add(msgs)
        return pl.pallas_call(
            _bias_kernel,
            out_shape=jax.ShapeDtypeStruct(out.shape, out.dtype),
        )(out, jnp.broadcast_to(b[None, :], out.shape))

    h = jax.nn.relu(conv(x, W1, b1))
    return conv(h, W2, b2)


# speedup vs baseline: 8.9148x; 8.2336x over previous
import functools

import jax
import jax.numpy as jnp
from jax import lax
from jax.experimental import pallas as pl
from jax.experimental.pallas import tpu as pltpu
from jax.experimental.pallas import tpu_sc as plsc

_NC = 2
_NS = 16
_N = 10000
_E = 320000
_D = 128
_HD = _D // _NC
_CH = 160
_ES = _CH * 128
_ACC = 10240
_RPS = _ACC // _NS
_ZR = _RPS // 128
_DCH = _CH // _NC


def _deg_body(eip_hbm, out_hbm, eib, idxb, ones_v, acc, sem):
    c = lax.axis_index("c")
    s = lax.axis_index("s")
    sync = pltpu.sync_copy
    for r in range(128):
        ones_v[r, pl.ds(0, 16)] = jnp.zeros((16,), jnp.float32)
    for i in range(_ZR):
        sync(ones_v, acc.at[pl.ds(s * _RPS + i * 128, 128)])
    onerow = jnp.where(lax.iota(jnp.int32, 16) < 1, 1.0, 0.0)
    for r in range(128):
        ones_v[r, pl.ds(0, 16)] = onerow

    sync(eip_hbm.at[s, pl.ds(c * _DCH, _DCH)], eib)

    def unpack(r, carry):
        for k in range(8):
            idxb[r, pl.ds(k * 16, 16)] = eib[r, pl.ds(k * 16, 16)] >> 14
        return carry

    lax.fori_loop(0, _DCH, unpack, 0)
    plsc.subcore_barrier()

    def chunk(j, carry):
        sync(ones_v, acc.at[idxb.at[j]], add=True)
        return carry

    lax.fori_loop(0, _DCH, chunk, 0)
    plsc.subcore_barrier()
    sync(acc.at[pl.ds(s * _RPS, _RPS)], out_hbm.at[c, pl.ds(s * _RPS, _RPS)])


def _agg_body(m_hbm, eip_hbm, out_hbm, eib, srcb, dstb, rows, acc, sem0, sem1):
    c = lax.axis_index("c")
    s = lax.axis_index("s")
    sync = pltpu.sync_copy

    z = jnp.zeros((16,), jnp.float32)

    def zrow(r, carry):
        for k in range(_HD // 16):
            rows[0, r, pl.ds(k * 16, 16)] = z
        return carry

    lax.fori_loop(0, 128, zrow, 0)
    for i in range(_ZR):
        sync(rows.at[0], acc.at[pl.ds(s * _RPS + i * 128, 128)])
    plsc.subcore_barrier()

    sync(eip_hbm.at[s], eib)

    def unpack(r, carry):
        for k in range(8):
            v = eib[r, pl.ds(k * 16, 16)]
            srcb[r, pl.ds(k * 16, 16)] = ((v & 16383) << 1) | c
            dstb[r, pl.ds(k * 16, 16)] = v >> 14
        return carry

    lax.fori_loop(0, _CH, unpack, 0)
    for r in (_CH, _CH + 1):
        for k in range(8):
            srcb[r, pl.ds(k * 16, 16)] = jnp.zeros((16,), jnp.int32)

    pltpu.async_copy(m_hbm.at[srcb.at[0]], rows.at[0], sem0)

    def step(t, carry):
        j0 = 2 * t
        pltpu.async_copy(m_hbm.at[srcb.at[j0 + 1]], rows.at[1], sem1)
        pltpu.make_async_copy(m_hbm.at[srcb.at[j0]], rows.at[0], sem0).wait()
        sync(rows.at[0], acc.at[dstb.at[j0]], add=True)
        pltpu.async_copy(m_hbm.at[srcb.at[j0 + 2]], rows.at[0], sem0)
        pltpu.make_async_copy(m_hbm.at[srcb.at[j0 + 1]], rows.at[1], sem1).wait()
        sync(rows.at[1], acc.at[dstb.at[j0 + 1]], add=True)
        return carry

    lax.fori_loop(0, _CH // 2, step, 0)
    pltpu.make_async_copy(m_hbm.at[srcb.at[0]], rows.at[0], sem0).wait()
    plsc.subcore_barrier()
    sync(acc.at[pl.ds(s * _RPS, _RPS)], out_hbm.at[pl.ds(s * _RPS, _RPS), c])


_sc_mesh = plsc.VectorSubcoreMesh(core_axis_name="c", subcore_axis_name="s")

_deg_kernel = functools.partial(
    pl.kernel,
    out_type=jax.ShapeDtypeStruct((_NC, _ACC, 16), jnp.float32),
    mesh=_sc_mesh,
    scratch_types=[
        pltpu.VMEM((_DCH, 128), jnp.int32),
        pltpu.VMEM((_DCH, 128), jnp.int32),
        pltpu.VMEM((128, 16), jnp.float32),
        pltpu.VMEM_SHARED((_ACC, 16), jnp.float32),
        pltpu.SemaphoreType.DMA,
    ],
)(_deg_body)

_agg_kernel = functools.partial(
    pl.kernel,
    out_type=jax.ShapeDtypeStruct((_ACC, _NC, _HD), jnp.float32),
    mesh=_sc_mesh,
    scratch_types=[
        pltpu.VMEM((_CH, 128), jnp.int32),
        pltpu.VMEM((_CH + 2, 128), jnp.int32),
        pltpu.VMEM((_CH, 128), jnp.int32),
        pltpu.VMEM((2, 128, _HD), jnp.float32),
        pltpu.VMEM_SHARED((_ACC, _HD), jnp.float32),
        pltpu.SemaphoreType.DMA,
        pltpu.SemaphoreType.DMA,
    ],
    compiler_params=pltpu.CompilerParams(use_tc_tiling_on_sc=False),
)(_agg_body)


def _dis(dp_ref):
    deg = dp_ref[0, :, 0:1] + dp_ref[1, :, 0:1] + 1.0
    return lax.rsqrt(deg)


def _mm1_body(dp_ref, x_ref, w_ref, o_ref):
    m = jnp.dot(x_ref[...], w_ref[...], preferred_element_type=jnp.float32)
    o_ref[...] = m * _dis(dp_ref)


def _mid_body(dp_ref, p_ref, m1_ref, b1_ref, w2_ref, o_ref):
    dis = _dis(dp_ref)
    a1 = (p_ref[...] + m1_ref[...]) * dis
    h = jnp.maximum(a1 + b1_ref[...], 0.0)
    o_ref[...] = jnp.dot(h, w2_ref[...], preferred_element_type=jnp.float32) * dis


def _fin_body(dp_ref, q_ref, m2_ref, b2_ref, o_ref):
    o_ref[...] = (q_ref[...] + m2_ref[...]) * _dis(dp_ref) + b2_ref[...]


_BLK = 1000
_GRID = _N // _BLK

_dp_spec = pl.BlockSpec((_NC, _BLK, 16), lambda i: (0, i, 0))
_row_spec = pl.BlockSpec((_BLK, _D), lambda i: (i, 0))
_w_spec = pl.BlockSpec((_D, _D), lambda i: (0, 0))
_b_spec = pl.BlockSpec((1, _D), lambda i: (0, 0))
_out_sds = jax.ShapeDtypeStruct((_N, _D), jnp.float32)


def kernel(x, edge_index, W1, b1, W2, b2):
    src = edge_index[0].astype(jnp.int32)
    dst = edge_index[1].astype(jnp.int32)
    pad = _ES - _E // _NS
    eip = jnp.pad(
        (src | (dst << 14)).reshape(_NS, _E // _NS),
        ((0, 0), (0, pad)),
        constant_values=_N << 14,
    ).reshape(_NS, _CH, 128)

    degp = _deg_kernel(eip)

    m1 = pl.pallas_call(
        _mm1_body,
        grid=(_GRID,),
        in_specs=[_dp_spec, _row_spec, _w_spec],
        out_specs=_row_spec,
        out_shape=_out_sds,
    )(degp, x, W1)

    p = _agg_kernel(m1.reshape(_NC * _N, _HD), eip).reshape(_ACC, _D)

    m2 = pl.pallas_call(
        _mid_body,
        grid=(_GRID,),
        in_specs=[_dp_spec, _row_spec, _row_spec, _b_spec, _w_spec],
        out_specs=_row_spec,
        out_shape=_out_sds,
    )(degp, p, m1, b1.reshape(1, _D), W2)

    q = _agg_kernel(m2.reshape(_NC * _N, _HD), eip).reshape(_ACC, _D)

    return pl.pallas_call(
        _fin_body,
        grid=(_GRID,),
        in_specs=[_dp_spec, _row_spec, _row_spec, _b_spec],
        out_specs=_row_spec,
        out_shape=_out_sds,
    )(degp, q, m2, b2.reshape(1, _D))
